# baseline (speedup 1.0000x reference)
import jax
import jax.numpy as jnp
from functools import partial
from jax.experimental import pallas as pl
from jax.experimental.pallas import tpu as pltpu

N = 10000
L = 128
DM = 32
DI = 32
DS = 8
DC = 4
DTR = 2
HID = 128

MB = 80
CB = 1000


def _silu(v):
    return v * jax.nn.sigmoid(v)


def _mamba_body(xT_ref, ipw_ref, cw_ref, cb_ref, xpw_ref, dtw_ref, dtb_ref,
                Alog_ref, Dp_ref, wsum_ref, out_ref,
                dl_ref, bm_ref, cm_ref, xc_ref, g_ref, s_ref):
    xb = xT_ref[...]
    Lb, B, _ = xb.shape
    m = xb.reshape(Lb * B, DM)
    xz = jnp.dot(m, ipw_ref[...], preferred_element_type=jnp.float32)
    xi = xz[:, :DI].reshape(Lb, B, DI)
    z = xz[:, DI:].reshape(Lb, B, DI)
    g_ref[...] = _silu(z)

    cw = cw_ref[...]
    pad = jnp.concatenate([jnp.zeros((DC - 1, B, DI), xb.dtype), xi], axis=0)
    conv = jnp.broadcast_to(cb_ref[...].reshape(1, 1, DI), (Lb, B, DI))
    for k in range(DC):
        conv = conv + pad[k:k + Lb] * cw[:, k].reshape(1, 1, DI)
    xc = _silu(conv)
    xc_ref[...] = xc

    xdbl = jnp.dot(xc.reshape(Lb * B, DI), xpw_ref[...],
                   preferred_element_type=jnp.float32)
    dt = xdbl[:, :DTR]
    bm_ref[...] = xdbl[:, DTR:DTR + DS].reshape(Lb, B, DS)
    cm_ref[...] = xdbl[:, DTR + DS:].reshape(Lb, B, DS)
    dtw = dtw_ref[...]
    dpre = (dt[:, 0:1] * dtw[0:1, :] + dt[:, 1:2] * dtw[1:2, :]
            + dtb_ref[...].reshape(1, DI))
    dl_ref[...] = jax.nn.softplus(dpre).reshape(Lb, B, DI)

    AT = -jnp.exp(Alog_ref[...]).T
    Dp = Dp_ref[...].reshape(1, DI)
    wsum = wsum_ref[...].reshape(1, DI)

    def step(t, h):
        d_t = dl_ref[pl.ds(t, 1)][0]
        b_t = bm_ref[pl.ds(t, 1)][0]
        c_t = cm_ref[pl.ds(t, 1)][0]
        u_t = xc_ref[pl.ds(t, 1)][0]
        g_t = g_ref[pl.ds(t, 1)][0]
        dA = jnp.exp(d_t[:, None, :] * AT[None, :, :])
        h = dA * h + (d_t * u_t)[:, None, :] * b_t[:, :, None]
        y = jnp.sum(h * c_t[:, :, None], axis=1)
        yf = (y + u_t * Dp) * g_t
        s = jnp.sum(yf * wsum, axis=1)
        s_ref[pl.ds(t, 1)] = s[None, :]
        return h

    h0 = jnp.zeros((B, DS, DI), xb.dtype)
    jax.lax.fori_loop(0, Lb, step, h0)
    out_ref[...] = s_ref[...].T


def _ln_body(sa_ref, sbf_ref, g_ref, b_ref, o_ref):
    s = jnp.maximum(sa_ref[...] + sbf_ref[...], 0.0)
    mu = jnp.mean(s, axis=1, keepdims=True)
    var = jnp.mean((s - mu) ** 2, axis=1, keepdims=True)
    o_ref[...] = ((s - mu) * jax.lax.rsqrt(var + 1e-5) * g_ref[...]
                  + b_ref[...])


def _comb_body(p_ref, q_ref, x_ref, wi_ref, wo_ref, wl_ref, b_ref, o_ref, *,
               relu):
    acc = 0.5 * jnp.dot(q_ref[...], wo_ref[...],
                        preferred_element_type=jnp.float32)
    acc = acc + 0.5 * jnp.dot(p_ref[...], wi_ref[...],
                              preferred_element_type=jnp.float32)
    acc = acc + jnp.dot(x_ref[...], wl_ref[...],
                        preferred_element_type=jnp.float32)
    acc = acc + b_ref[...]
    o_ref[...] = jnp.maximum(acc, 0.0) if relu else acc


def _full(shape):
    return pl.BlockSpec(shape, lambda i: tuple(0 for _ in shape))


def _mamba_call(xT, ipw, cw, cb2, xpw, dtw, dtb2, Alog, Dp2, wsum2):
    n2 = xT.shape[1]
    grid = n2 // MB
    f32 = jnp.float32
    return pl.pallas_call(
        _mamba_body,
        grid=(grid,),
        in_specs=[
            pl.BlockSpec((L, MB, DM), lambda i: (0, i, 0)),
            _full(ipw.shape), _full(cw.shape), _full(cb2.shape),
            _full(xpw.shape), _full(dtw.shape), _full(dtb2.shape),
            _full(Alog.shape), _full(Dp2.shape), _full(wsum2.shape),
        ],
        out_specs=pl.BlockSpec((MB, L), lambda i: (i, 0)),
        out_shape=jax.ShapeDtypeStruct((n2, L), f32),
        scratch_shapes=[
            pltpu.VMEM((L, MB, DI), f32),
            pltpu.VMEM((L, MB, DS), f32),
            pltpu.VMEM((L, MB, DS), f32),
            pltpu.VMEM((L, MB, DI), f32),
            pltpu.VMEM((L, MB, DI), f32),
            pltpu.VMEM((L, MB), f32),
        ],
    )(xT, ipw, cw, cb2, xpw, dtw, dtb2, Alog, Dp2, wsum2)


def _ln_call(sa, sbf, g2, b2):
    return pl.pallas_call(
        _ln_body,
        grid=(N // CB,),
        in_specs=[
            pl.BlockSpec((CB, L), lambda i: (i, 0)),
            pl.BlockSpec((CB, L), lambda i: (i, 0)),
            _full(g2.shape), _full(b2.shape),
        ],
        out_specs=pl.BlockSpec((CB, L), lambda i: (i, 0)),
        out_shape=jax.ShapeDtypeStruct((N, L), jnp.float32),
    )(sa, sbf, g2, b2)


def _comb_call(p, q, xin, wi, wo, wl, b2, relu):
    return pl.pallas_call(
        partial(_comb_body, relu=relu),
        grid=(N // CB,),
        in_specs=[
            pl.BlockSpec((CB, HID), lambda i: (i, 0)),
            pl.BlockSpec((CB, HID), lambda i: (i, 0)),
            pl.BlockSpec((CB, HID), lambda i: (i, 0)),
            _full(wi.shape), _full(wo.shape), _full(wl.shape), _full(b2.shape),
        ],
        out_specs=pl.BlockSpec((CB, HID), lambda i: (i, 0)),
        out_shape=jax.ShapeDtypeStruct((N, HID), jnp.float32),
    )(p, q, xin, wi, wo, wl, b2)


def kernel(x, input_ids, attention_mask, edge_index, in_proj_w, conv1d_w,
           conv1d_b, x_proj_w, dt_proj_w, dt_proj_b, A_log, D_p, out_proj_w,
           ln_g, ln_b, W_in1, b_in1, W_out1, b_out1, W_lin1, b_lin1,
           W_in2, b_in2, W_out2, b_out2, W_lin2, b_lin2):
    f32 = jnp.float32
    x2 = jnp.concatenate([x, x[:, ::-1, :]], axis=0)
    xT = jnp.transpose(x2, (1, 0, 2))

    wsum2 = jnp.sum(out_proj_w, axis=1).reshape(1, DI)
    sf = _mamba_call(xT, in_proj_w, conv1d_w, conv1d_b.reshape(1, DI),
                     x_proj_w, dt_proj_w, dt_proj_b.reshape(1, DI),
                     A_log, D_p.reshape(1, DI), wsum2)

    sa = sf[:N]
    sbf = sf[N:, ::-1]
    sn = _ln_call(sa, sbf, ln_g.reshape(1, L), ln_b.reshape(1, L))

    src = edge_index[0]
    dst = edge_index[1]
    deg_in = jnp.zeros((N,), f32).at[dst].add(1.0)
    deg_out = jnp.zeros((N,), f32).at[src].add(1.0)
    di = jnp.where(deg_in > 0, jax.lax.rsqrt(jnp.maximum(deg_in, 1e-12)), 0.0)
    do = jnp.where(deg_out > 0, jax.lax.rsqrt(jnp.maximum(deg_out, 1e-12)),
                   0.0)
    nf = di[src] * di[dst]
    nb = do[src] * do[dst]

    def agg(v):
        p = jnp.zeros_like(v).at[dst].add(nf[:, None] * v[src])
        q = jnp.zeros_like(v).at[src].add(nb[:, None] * v[dst])
        return p, q

    btot1 = (0.5 * b_in1 + 0.5 * b_out1 + b_lin1).reshape(1, HID)
    btot2 = (0.5 * b_in2 + 0.5 * b_out2 + b_lin2).reshape(1, HID)

    p1, q1 = agg(sn)
    h1 = _comb_call(p1, q1, sn, W_in1, W_out1, W_lin1, btot1, True)
    p2, q2 = agg(h1)
    out = _comb_call(p2, q2, h1, W_in2, W_out2, W_lin2, btot2, False)
    return out


# speedup vs baseline: 1.1652x; 1.1652x over previous
import jax
import jax.numpy as jnp
from functools import partial
from jax.experimental import pallas as pl
from jax.experimental.pallas import tpu as pltpu

N = 10000
L = 128
DM = 32
DI = 32
DS = 8
DC = 4
DTR = 2
HID = 128

MB = 80
CB = 1000


def _silu(v):
    return v * jax.nn.sigmoid(v)


def _mamba_body(xT_ref, ipw_ref, cw_ref, cb_ref, xpw_ref, dtw_ref, dtb_ref,
                Alog_ref, Dp_ref, wsum_ref, out_ref,
                dl_ref, bm_ref, cm_ref, xc_ref, g_ref, y_ref):
    xb = xT_ref[...]
    Lb, B, _ = xb.shape
    m = xb.reshape(Lb * B, DM)
    xz = jnp.dot(m, ipw_ref[...], preferred_element_type=jnp.float32)
    xi = xz[:, :DI].reshape(Lb, B, DI)
    z = xz[:, DI:].reshape(Lb, B, DI)
    g_ref[...] = _silu(z)

    cw = cw_ref[...]
    pad = jnp.concatenate([jnp.zeros((DC - 1, B, DI), xb.dtype), xi], axis=0)
    conv = jnp.broadcast_to(cb_ref[...].reshape(1, 1, DI), (Lb, B, DI))
    for k in range(DC):
        conv = conv + pad[k:k + Lb] * cw[:, k].reshape(1, 1, DI)
    xc = _silu(conv)
    xc_ref[...] = xc

    xdbl = jnp.dot(xc.reshape(Lb * B, DI), xpw_ref[...],
                   preferred_element_type=jnp.float32)
    dt = xdbl[:, :DTR]
    bm_ref[...] = xdbl[:, DTR:DTR + DS].reshape(Lb, B, DS)
    cm_ref[...] = xdbl[:, DTR + DS:].reshape(Lb, B, DS)
    dtw = dtw_ref[...]
    dpre = (dt[:, 0:1] * dtw[0:1, :] + dt[:, 1:2] * dtw[1:2, :]
            + dtb_ref[...].reshape(1, DI))
    dl_ref[...] = jax.nn.softplus(dpre).reshape(Lb, B, DI)

    AT = -jnp.exp(Alog_ref[...]).T
    Dp = Dp_ref[...].reshape(1, DI)
    wsum = wsum_ref[...].reshape(1, DI)

    def step(t, h):
        d_t = dl_ref[pl.ds(t, 1)][0]
        b_t = bm_ref[pl.ds(t, 1)][0]
        c_t = cm_ref[pl.ds(t, 1)][0]
        u_t = xc_ref[pl.ds(t, 1)][0]
        dA = jnp.exp(d_t[:, None, :] * AT[None, :, :])
        h = dA * h + (d_t * u_t)[:, None, :] * b_t[:, :, None]
        y = jnp.sum(h * c_t[:, :, None], axis=1)
        y_ref[pl.ds(t, 1)] = y[None, :, :]
        return h

    h0 = jnp.zeros((B, DS, DI), xb.dtype)
    jax.lax.fori_loop(0, Lb, step, h0)
    yb = y_ref[...]
    s = jnp.sum((yb + xc_ref[...] * Dp[None]) * g_ref[...]
                * wsum[None], axis=2)
    out_ref[...] = s.T


def _ln_body(sa_ref, sbf_ref, g_ref, b_ref, o_ref):
    s = jnp.maximum(sa_ref[...] + sbf_ref[...], 0.0)
    mu = jnp.mean(s, axis=1, keepdims=True)
    var = jnp.mean((s - mu) ** 2, axis=1, keepdims=True)
    o_ref[...] = ((s - mu) * jax.lax.rsqrt(var + 1e-5) * g_ref[...]
                  + b_ref[...])


def _comb_body(p_ref, q_ref, x_ref, wi_ref, wo_ref, wl_ref, b_ref, o_ref, *,
               relu):
    acc = 0.5 * jnp.dot(q_ref[...], wo_ref[...],
                        preferred_element_type=jnp.float32)
    acc = acc + 0.5 * jnp.dot(p_ref[...], wi_ref[...],
                              preferred_element_type=jnp.float32)
    acc = acc + jnp.dot(x_ref[...], wl_ref[...],
                        preferred_element_type=jnp.float32)
    acc = acc + b_ref[...]
    o_ref[...] = jnp.maximum(acc, 0.0) if relu else acc


def _full(shape):
    return pl.BlockSpec(shape, lambda i: tuple(0 for _ in shape))


def _mamba_call(xT, ipw, cw, cb2, xpw, dtw, dtb2, Alog, Dp2, wsum2):
    n2 = xT.shape[1]
    grid = n2 // MB
    f32 = jnp.float32
    return pl.pallas_call(
        _mamba_body,
        grid=(grid,),
        in_specs=[
            pl.BlockSpec((L, MB, DM), lambda i: (0, i, 0)),
            _full(ipw.shape), _full(cw.shape), _full(cb2.shape),
            _full(xpw.shape), _full(dtw.shape), _full(dtb2.shape),
            _full(Alog.shape), _full(Dp2.shape), _full(wsum2.shape),
        ],
        out_specs=pl.BlockSpec((MB, L), lambda i: (i, 0)),
        out_shape=jax.ShapeDtypeStruct((n2, L), f32),
        scratch_shapes=[
            pltpu.VMEM((L, MB, DI), f32),
            pltpu.VMEM((L, MB, DS), f32),
            pltpu.VMEM((L, MB, DS), f32),
            pltpu.VMEM((L, MB, DI), f32),
            pltpu.VMEM((L, MB, DI), f32),
            pltpu.VMEM((L, MB, DI), f32),
        ],
    )(xT, ipw, cw, cb2, xpw, dtw, dtb2, Alog, Dp2, wsum2)


def _ln_call(sa, sbf, g2, b2):
    return pl.pallas_call(
        _ln_body,
        grid=(N // CB,),
        in_specs=[
            pl.BlockSpec((CB, L), lambda i: (i, 0)),
            pl.BlockSpec((CB, L), lambda i: (i, 0)),
            _full(g2.shape), _full(b2.shape),
        ],
        out_specs=pl.BlockSpec((CB, L), lambda i: (i, 0)),
        out_shape=jax.ShapeDtypeStruct((N, L), jnp.float32),
    )(sa, sbf, g2, b2)


def _comb_call(p, q, xin, wi, wo, wl, b2, relu):
    return pl.pallas_call(
        partial(_comb_body, relu=relu),
        grid=(N // CB,),
        in_specs=[
            pl.BlockSpec((CB, HID), lambda i: (i, 0)),
            pl.BlockSpec((CB, HID), lambda i: (i, 0)),
            pl.BlockSpec((CB, HID), lambda i: (i, 0)),
            _full(wi.shape), _full(wo.shape), _full(wl.shape), _full(b2.shape),
        ],
        out_specs=pl.BlockSpec((CB, HID), lambda i: (i, 0)),
        out_shape=jax.ShapeDtypeStruct((N, HID), jnp.float32),
    )(p, q, xin, wi, wo, wl, b2)


def kernel(x, input_ids, attention_mask, edge_index, in_proj_w, conv1d_w,
           conv1d_b, x_proj_w, dt_proj_w, dt_proj_b, A_log, D_p, out_proj_w,
           ln_g, ln_b, W_in1, b_in1, W_out1, b_out1, W_lin1, b_lin1,
           W_in2, b_in2, W_out2, b_out2, W_lin2, b_lin2):
    f32 = jnp.float32
    x2 = jnp.concatenate([x, x[:, ::-1, :]], axis=0)
    xT = jnp.transpose(x2, (1, 0, 2))

    wsum2 = jnp.sum(out_proj_w, axis=1).reshape(1, DI)
    sf = _mamba_call(xT, in_proj_w, conv1d_w, conv1d_b.reshape(1, DI),
                     x_proj_w, dt_proj_w, dt_proj_b.reshape(1, DI),
                     A_log, D_p.reshape(1, DI), wsum2)

    sa = sf[:N]
    sbf = sf[N:, ::-1]
    sn = _ln_call(sa, sbf, ln_g.reshape(1, L), ln_b.reshape(1, L))

    src = edge_index[0]
    dst = edge_index[1]
    deg_in = jnp.zeros((N,), f32).at[dst].add(1.0)
    deg_out = jnp.zeros((N,), f32).at[src].add(1.0)
    di = jnp.where(deg_in > 0, jax.lax.rsqrt(jnp.maximum(deg_in, 1e-12)), 0.0)
    do = jnp.where(deg_out > 0, jax.lax.rsqrt(jnp.maximum(deg_out, 1e-12)),
                   0.0)
    nf = di[src] * di[dst]
    nb = do[src] * do[dst]

    def agg(v):
        p = jnp.zeros_like(v).at[dst].add(nf[:, None] * v[src])
        q = jnp.zeros_like(v).at[src].add(nb[:, None] * v[dst])
        return p, q

    btot1 = (0.5 * b_in1 + 0.5 * b_out1 + b_lin1).reshape(1, HID)
    btot2 = (0.5 * b_in2 + 0.5 * b_out2 + b_lin2).reshape(1, HID)

    p1, q1 = agg(sn)
    h1 = _comb_call(p1, q1, sn, W_in1, W_out1, W_lin1, btot1, True)
    p2, q2 = agg(h1)
    out = _comb_call(p2, q2, h1, W_in2, W_out2, W_lin2, btot2, False)
    return out
